# baseline (speedup 1.0000x reference)
import functools

import jax
import jax.numpy as jnp
from jax import lax
from jax.experimental import pallas as pl
from jax.experimental.pallas import tpu as pltpu
from jax.experimental.pallas import tpu_sc as plsc

N = 10000
E = 320000
D = 128
DP = 64
DA = 160
NC = 2
NS = 16
NP = 10240
EPT = E // (NC * NS)
K = 80
CH = EPT // K
RPT = NP // NS
ZR = 40
NEG = 0.2
L = 16


def _tc_prep_body(x_ref, w_ref, a2_ref, hp_ref, sd_ref):
    h = jnp.dot(x_ref[...], w_ref[...], preferred_element_type=jnp.float32)
    hp_ref[...] = h.astype(jnp.bfloat16)
    sd_ref[...] = lax.dot_general(
        a2_ref[...], h, (((1,), (1,)), ((), ())),
        preferred_element_type=jnp.float32)


def _tc_prep(x, w, a2):
    return pl.pallas_call(
        _tc_prep_body,
        out_shape=[
            jax.ShapeDtypeStruct((N, D), jnp.bfloat16),
            jax.ShapeDtypeStruct((2, N), jnp.float32),
        ],
    )(x, w, a2)


def _combine(p_ref, b_ref):
    feat = (p_ref[0, :N, :D].astype(jnp.float32)
            + p_ref[1, :N, :D].astype(jnp.float32))
    denom = (p_ref[0, :N, D:D + 1].astype(jnp.float32)
             + p_ref[1, :N, D:D + 1].astype(jnp.float32))
    denom = jnp.where(denom == 0.0, 1.0, denom)
    return feat / denom + b_ref[...]


def _tc_mid_body(p_ref, b_ref, w_ref, a2_ref, hp_ref, sd_ref):
    h1 = jnp.maximum(_combine(p_ref, b_ref), 0.0)
    h2 = jnp.dot(h1, w_ref[...], preferred_element_type=jnp.float32)
    hp_ref[...] = h2.astype(jnp.bfloat16)
    sd_ref[...] = lax.dot_general(
        a2_ref[...], h2, (((1,), (1,)), ((), ())),
        preferred_element_type=jnp.float32)


def _tc_mid(p, b, w, a2):
    return pl.pallas_call(
        _tc_mid_body,
        out_shape=[
            jax.ShapeDtypeStruct((N, D), jnp.bfloat16),
            jax.ShapeDtypeStruct((2, N), jnp.float32),
        ],
    )(p, b, w, a2)


def _tc_fin_body(p_ref, b_ref, o_ref):
    o_ref[...] = _combine(p_ref, b_ref)


def _tc_fin(p, b):
    return pl.pallas_call(
        _tc_fin_body,
        out_shape=jax.ShapeDtypeStruct((N, D), jnp.float32),
    )(p, b)


_mesh = plsc.VectorSubcoreMesh(core_axis_name="c", subcore_axis_name="s", num_cores=NC)


@functools.partial(
    pl.kernel,
    out_type=jax.ShapeDtypeStruct((NC, NP, DA), jnp.bfloat16),
    mesh=_mesh,
    scratch_types=[
        pltpu.VMEM((N,), jnp.float32),
        pltpu.VMEM((N,), jnp.float32),
        pltpu.VMEM((EPT,), jnp.int32),
        pltpu.VMEM((EPT,), jnp.int32),
        pltpu.VMEM((K,), jnp.int32),
        pltpu.VMEM((K,), jnp.int32),
        pltpu.VMEM((K,), jnp.float32),
        pltpu.VMEM((K, D), jnp.bfloat16),
        pltpu.VMEM((K, D), jnp.bfloat16),
        pltpu.VMEM((K, DA), jnp.bfloat16),
        pltpu.VMEM((K, DA), jnp.bfloat16),
        pltpu.VMEM((ZR, DA), jnp.bfloat16),
        pltpu.VMEM_SHARED((NP, DA), jnp.bfloat16),
        pltpu.SemaphoreType.DMA,
        pltpu.SemaphoreType.DMA,
        pltpu.SemaphoreType.DMA,
        pltpu.SemaphoreType.DMA,
    ],
    compiler_params=pltpu.CompilerParams(needs_layout_passes=False, use_tc_tiling_on_sc=False),
)
def _sc_edge(hp_hbm, sd_hbm, src_hbm, dst_hbm, out_hbm,
             s_t, d_t, src_all, dst_all, dstm0, dstm1, w_buf,
             rows_g0, rows_g1, rows_s0, rows_s1, zbuf,
             acc, gsem0, gsem1, ssem0, ssem1):
    cid = lax.axis_index("c")
    sid = lax.axis_index("s")
    dstm = (dstm0, dstm1)
    rows_g = (rows_g0, rows_g1)
    rows_s = (rows_s0, rows_s1)
    gsem = (gsem0, gsem1)
    ssem = (ssem0, ssem1)

    pltpu.sync_copy(sd_hbm.at[0], s_t)
    pltpu.sync_copy(sd_hbm.at[1], d_t)
    ebase = (cid * NS + sid) * EPT
    pltpu.sync_copy(src_hbm.at[pl.ds(ebase, EPT)], src_all)
    pltpu.sync_copy(dst_hbm.at[pl.ds(ebase, EPT)], dst_all)

    zb16 = jnp.zeros((2 * L,), jnp.bfloat16)

    def _zero_row(r, _):
        for j in range(DA // (2 * L)):
            zbuf[r, pl.ds(j * 2 * L, 2 * L)] = zb16
        return 0
    lax.fori_loop(0, ZR, _zero_row, 0)
    for part in range(RPT // ZR):
        pltpu.sync_copy(zbuf, acc.at[pl.ds(sid * RPT + part * ZR, ZR)])

    def _max_body(i, carry):
        ms, md = carry
        ms = jnp.maximum(ms, s_t[pl.ds(i * L, L)])
        md = jnp.maximum(md, d_t[pl.ds(i * L, L)])
        return ms, md
    ninf = jnp.full((L,), -jnp.inf, jnp.float32)
    ms, md = lax.fori_loop(0, N // L, _max_body, (ninf, ninf))
    lanes = lax.iota(jnp.int32, L)
    for sh in (8, 4, 2, 1):
        perm = lanes ^ sh
        ms = jnp.maximum(ms, ms.at[perm].get(mode="promise_in_bounds"))
        md = jnp.maximum(md, md.at[perm].get(mode="promise_in_bounds"))
    mv = ms + md
    mvec = jnp.where(mv > 0.0, mv, NEG * mv)

    onehot = jnp.where(
        lanes == 0,
        jnp.ones((L,), jnp.float32), jnp.zeros((L,), jnp.float32))
    zf = jnp.zeros((L,), jnp.float32)

    plsc.subcore_barrier()

    def _chunk(c, b, first, mvec):
        pltpu.make_async_copy(hp_hbm.at[src_all.at[pl.ds(c * K, K)]],
                              rows_g[b], gsem[b]).wait()
        if not first:
            pltpu.make_async_copy(rows_s[b], acc.at[dstm[b]], ssem[b]).wait()
        for q in range(K // L):
            si = src_all[pl.ds(c * K + q * L, L)]
            di = dst_all[pl.ds(c * K + q * L, L)]
            e = plsc.load_gather(s_t, [si]) + plsc.load_gather(d_t, [di])
            e = jnp.where(e > 0.0, e, NEG * e)
            w_buf[pl.ds(q * L, L)] = jnp.exp(e - mvec)
            dstm[b][pl.ds(q * L, L)] = di
        def _scale(q, _):
            wv16 = w_buf[pl.ds(q * L, L)]
            for u in range(L):
                i = q * L + u
                wv = wv16.at[jnp.full((L,), u, jnp.int32)].get(
                    mode="promise_in_bounds")
                for j in range(DP // L):
                    words = rows_g[b][i, pl.ds(j * 2 * L, 2 * L)]
                    pair = plsc.unpack(words,
                                       format=plsc.PackFormat.INTERLEAVED)
                    lo = pair[0].astype(jnp.float32) * wv
                    hi = pair[1].astype(jnp.float32) * wv
                    rows_s[b][i, pl.ds(j * 2 * L, 2 * L)] = plsc.pack(
                        lo, hi, format=plsc.PackFormat.INTERLEAVED)
                rows_s[b][i, pl.ds(D, 2 * L)] = plsc.pack(
                    wv * onehot, zf, format=plsc.PackFormat.INTERLEAVED)
            return 0
        lax.fori_loop(0, K // L, _scale, 0)
        pltpu.async_copy(rows_s[b], acc.at[dstm[b]], ssem[b], add=True)
        @pl.when(c + 2 < CH)
        def _():
            pltpu.async_copy(hp_hbm.at[src_all.at[pl.ds((c + 2) * K, K)]],
                             rows_g[b], gsem[b])
        return mvec

    for b in range(2):
        pltpu.async_copy(hp_hbm.at[src_all.at[pl.ds(b * K, K)]],
                         rows_g[b], gsem[b])

    def _pair(g, mvec):
        for b in range(2):
            mvec = _chunk(2 * g + b, b, False, mvec)
        return mvec

    mvec = _chunk(0, 0, True, mvec)
    mvec = _chunk(1, 1, True, mvec)
    mvec = lax.fori_loop(1, CH // 2, _pair, mvec)
    _chunk(CH - 1, 0, False, mvec)

    for b in range(2):
        pltpu.make_async_copy(rows_s[b], acc.at[dstm[b]], ssem[b]).wait()

    plsc.subcore_barrier()
    pltpu.sync_copy(acc.at[pl.ds(sid * RPT, RPT)],
                    out_hbm.at[cid, pl.ds(sid * RPT, RPT)])


def kernel(x, edge_index, W1, a_src1, a_dst1, b1, W2, a_src2, a_dst2, b2):
    src = edge_index[0]
    dst = edge_index[1]
    a21 = jnp.stack([a_src1, a_dst1])
    a22 = jnp.stack([a_src2, a_dst2])

    hp1, sd1 = _tc_prep(x, W1, a21)
    p1 = _sc_edge(hp1, sd1, src, dst)
    hp2, sd2 = _tc_mid(p1, b1.reshape(1, D), W2, a22)
    p2 = _sc_edge(hp2, sd2, src, dst)
    return _tc_fin(p2, b2.reshape(1, D))
